# Initial kernel scaffold; baseline (speedup 1.0000x reference)
import jax
import jax.numpy as jnp
from jax.experimental import pallas as pl


def kernel(X, params, graph_segment_ids, W_in, b_in, W_pred, b_pred):
    raise NotImplementedError("write your pallas kernel here")


# speedup vs baseline: 10.0287x; 10.0287x over previous
import jax
import jax.numpy as jnp
from jax.experimental import pallas as pl
from jax.experimental.pallas import tpu as pltpu

_N = 50000
_D_IN = 128
_HID = 256
_OUT = 128
_NSEG = 256
_BLK = 2048
_NBLK = 25
_NPAD = _NBLK * _BLK


def _body(x_ref, ids_ref, win_ref, bin_ref, wpred_ref, bpred_ref, out_ref,
          acc_ref, cnt_ref):
    i = pl.program_id(0)

    @pl.when(i == 0)
    def _init():
        acc_ref[...] = jnp.zeros_like(acc_ref)
        cnt_ref[...] = jnp.zeros_like(cnt_ref)

    ids = ids_ref[0, 0, :]
    onehot_t = (jax.lax.broadcasted_iota(jnp.int32, (_NSEG, _BLK), 0)
                == ids[None, :]).astype(jnp.float32)
    acc_ref[...] += jax.lax.dot_general(
        onehot_t, x_ref[...], (((1,), (0,)), ((), ())),
        preferred_element_type=jnp.float32)
    cnt_ref[...] += jnp.sum(onehot_t, axis=1, keepdims=True)

    @pl.when(i == _NBLK - 1)
    def _finish():
        hidden = jax.lax.dot_general(
            acc_ref[...], win_ref[...], (((1,), (0,)), ((), ())),
            preferred_element_type=jnp.float32)
        hidden += cnt_ref[...] * bin_ref[...]
        out_ref[...] = jax.lax.dot_general(
            hidden, wpred_ref[...], (((1,), (0,)), ((), ())),
            preferred_element_type=jnp.float32) + bpred_ref[...]


def kernel(X, params, graph_segment_ids, W_in, b_in, W_pred, b_pred):
    del params
    ids = graph_segment_ids.astype(jnp.int32)
    Xp = jnp.pad(X, ((0, _NPAD - _N), (0, 0)))
    idsp = jnp.pad(ids, (0, _NPAD - _N), constant_values=_NSEG)
    idsp = idsp.reshape(_NBLK, 1, _BLK)

    return pl.pallas_call(
        _body,
        grid=(_NBLK,),
        in_specs=[
            pl.BlockSpec((_BLK, _D_IN), lambda i: (i, 0)),
            pl.BlockSpec((1, 1, _BLK), lambda i: (i, 0, 0)),
            pl.BlockSpec((_D_IN, _HID), lambda i: (0, 0)),
            pl.BlockSpec((1, _HID), lambda i: (0, 0)),
            pl.BlockSpec((_HID, _OUT), lambda i: (0, 0)),
            pl.BlockSpec((1, _OUT), lambda i: (0, 0)),
        ],
        out_specs=pl.BlockSpec((_NSEG, _OUT), lambda i: (0, 0)),
        out_shape=jax.ShapeDtypeStruct((_NSEG, _OUT), jnp.float32),
        scratch_shapes=[
            pltpu.VMEM((_NSEG, _D_IN), jnp.float32),
            pltpu.VMEM((_NSEG, 1), jnp.float32),
        ],
        compiler_params=pltpu.CompilerParams(
            dimension_semantics=("arbitrary",)),
    )(Xp, idsp, W_in, b_in.reshape(1, _HID), W_pred, b_pred.reshape(1, _OUT))
